# baseline (speedup 1.0000x reference)
import functools

import jax
import jax.numpy as jnp
from jax import lax
from jax.experimental import pallas as pl
from jax.experimental.pallas import tpu as pltpu, tpu_sc as plsc

D = 1024
SCALE = float(D) ** 0.5
NC = 2
NS = 16
NW = NC * NS
LANES = 16
NBUF = 6
CHUNK = 16


def _make_emb_kernel(batch: int, seq: int):
    n_rows = batch * seq
    per_w = n_rows // NW
    w_per_b = seq // per_w
    nchunk = per_w // CHUNK
    ngrp = CHUNK // LANES

    mesh = plsc.VectorSubcoreMesh(
        core_axis_name="c", subcore_axis_name="s",
        num_cores=NC, num_subcores=NS,
    )

    @functools.partial(
        pl.kernel,
        out_type=jax.ShapeDtypeStruct((batch, seq, D), jnp.float32),
        mesh=mesh,
        compiler_params=pltpu.CompilerParams(needs_layout_passes=False),
        scratch_types=[
            pltpu.VMEM((per_w,), jnp.int32),
            [pltpu.VMEM((CHUNK, D), jnp.float32)] * NBUF,
            [pltpu.SemaphoreType.DMA] * NBUF,
            [pltpu.SemaphoreType.DMA] * NBUF,
        ],
    )
    def emb(idx_hbm, table_hbm, out_hbm, idx_v, bufs, gsems, osems):
        wid = lax.axis_index("s") * NC + lax.axis_index("c")
        b_i = wid // w_per_b
        s_base = (wid % w_per_b) * per_w
        pltpu.sync_copy(idx_hbm.at[b_i, pl.ds(s_base, per_w)], idx_v)

        def start_gather(c):
            p = c % NBUF
            return pltpu.async_copy(
                table_hbm.at[idx_v.at[pl.ds(c * CHUNK, CHUNK)]],
                bufs[p], gsems[p])

        def start_out(c):
            p = c % NBUF
            return pltpu.async_copy(
                bufs[p],
                out_hbm.at[b_i, pl.ds(s_base + c * CHUNK, CHUNK)],
                osems[p])

        def compute(c):
            buf = bufs[c % NBUF]
            for g in range(ngrp):
                base_r = g * LANES
                iv = idx_v[pl.ds(c * CHUNK + base_r, LANES)]
                sv = jnp.where(iv == 0, 0.0, SCALE).astype(jnp.float32)

                def row_body(rr, carry):
                    bc = jnp.take_along_axis(
                        sv, jnp.full((LANES,), rr, jnp.int32), axis=0)
                    r = base_r + rr
                    for j in range(D // LANES):
                        buf[r, pl.ds(j * LANES, LANES)] = (
                            buf[r, pl.ds(j * LANES, LANES)] * bc)
                    return carry

                lax.fori_loop(0, LANES, row_body, 0)

        pf = NBUF - 1
        ghandles = [None] * NBUF
        ohandles = [None] * NBUF
        for c in range(min(pf, nchunk)):
            ghandles[c % NBUF] = start_gather(c)
        for c in range(nchunk):
            p = c % NBUF
            ghandles[p].wait()
            if c + pf < nchunk:
                q = (c + pf) % NBUF
                if ohandles[q] is not None:
                    ohandles[q].wait()
                ghandles[q] = start_gather(c + pf)
            compute(c)
            ohandles[p] = start_out(c)
        for h in ohandles:
            if h is not None:
                h.wait()

    return emb


@jax.jit
def kernel(inputs, shared_weights):
    b, s = inputs.shape
    emb = _make_emb_kernel(b, s)
    return emb(inputs.astype(jnp.int32), shared_weights)
